# Initial kernel scaffold; baseline (speedup 1.0000x reference)
import jax
import jax.numpy as jnp
from jax.experimental import pallas as pl


def kernel(x, token_table, pos_table):
    raise NotImplementedError("write your pallas kernel here")


# speedup vs baseline: 2.1744x; 2.1744x over previous
import functools

import jax
import jax.numpy as jnp
from jax import lax
from jax.experimental import pallas as pl
from jax.experimental.pallas import tpu as pltpu
from jax.experimental.pallas import tpu_sc as plsc

NC = 2
NS = 16
NW = NC * NS
CHUNK = 128
MAXLEN = 200
EMBED = 64
BATCH = 4096
BFLAT = BATCH * MAXLEN
NCHUNKS = BFLAT // CHUNK
CPW = NCHUNKS // NW


def _emb_body(x2_hbm, tab_hbm, pos2_hbm, out_hbm, idx_v, pos_v, buf, sem):
    wid = lax.axis_index("s") * NC + lax.axis_index("c")
    row0 = wid * CPW
    flat0 = row0 * CHUNK

    pltpu.sync_copy(pos2_hbm, pos_v)
    pltpu.sync_copy(x2_hbm.at[pl.ds(row0, CPW)], idx_v)

    @pl.loop(0, CPW)
    def _(c):
        pltpu.async_copy(tab_hbm.at[idx_v.at[c]], buf, sem).wait()
        off = lax.rem(c * CHUNK, MAXLEN)

        @pl.loop(0, CHUNK)
        def _(r):
            pr = off + r
            for g in range(EMBED // 16):
                s = pl.ds(g * 16, 16)
                buf[r, s] = buf[r, s] + pos_v[pr, s]

        pltpu.sync_copy(buf, out_hbm.at[pl.ds(flat0 + c * CHUNK, CHUNK)])


@jax.jit
def kernel(x, token_table, pos_table):
    x2 = x.reshape(NCHUNKS, CHUNK).astype(jnp.int32)
    pos2 = jnp.concatenate([pos_table, pos_table], axis=0)

    mesh = plsc.VectorSubcoreMesh(core_axis_name="c", subcore_axis_name="s")
    run = pl.kernel(
        _emb_body,
        out_type=jax.ShapeDtypeStruct((BFLAT, EMBED), jnp.float32),
        mesh=mesh,
        scratch_types=[
            pltpu.VMEM((CPW, CHUNK), jnp.int32),
            pltpu.VMEM((2 * MAXLEN, EMBED), jnp.float32),
            pltpu.VMEM((CHUNK, EMBED), jnp.float32),
            pltpu.SemaphoreType.DMA,
        ],
        compiler_params=pltpu.CompilerParams(use_tc_tiling_on_sc=False),
    )
    out = run(x2, token_table, pos2)
    return out.reshape(BATCH, MAXLEN, EMBED)


# speedup vs baseline: 2.8376x; 1.3050x over previous
import functools

import jax
import jax.numpy as jnp
from jax import lax
from jax.experimental import pallas as pl
from jax.experimental.pallas import tpu as pltpu
from jax.experimental.pallas import tpu_sc as plsc

NC = 2
NS = 16
NW = NC * NS
CHUNK = 128
MAXLEN = 200
EMBED = 64
BATCH = 4096
BFLAT = BATCH * MAXLEN
NCHUNKS = BFLAT // CHUNK
CPW = NCHUNKS // NW


NBUF = 4


def _emb_body(x2_hbm, tab_hbm, pos2_hbm, out_hbm, idx_v, pos_v,
              b0, b1, b2, b3, g0, g1, g2, g3, o0, o1, o2, o3):
    bufs = (b0, b1, b2, b3)
    gsems = (g0, g1, g2, g3)
    osems = (o0, o1, o2, o3)

    wid = lax.axis_index("s") * NC + lax.axis_index("c")
    row0 = wid * CPW
    flat0 = row0 * CHUNK

    pltpu.sync_copy(pos2_hbm, pos_v)
    pltpu.sync_copy(x2_hbm.at[pl.ds(row0, CPW)], idx_v)

    def gstart(c, p):
        pltpu.make_async_copy(tab_hbm.at[idx_v.at[c]], bufs[p], gsems[p]).start()

    def gwait(c, p):
        pltpu.make_async_copy(tab_hbm.at[idx_v.at[c]], bufs[p], gsems[p]).wait()

    def odesc(c, p):
        dst = out_hbm.at[pl.ds(flat0 + c * CHUNK, CHUNK)]
        return pltpu.make_async_copy(bufs[p], dst, osems[p])

    for p in range(NBUF - 1):
        gstart(p, p)

    @pl.loop(0, CPW, step=NBUF)
    def _(c):
        for k in range(NBUF):
            ck = c + k
            p = k
            pn = (k + NBUF - 1) % NBUF

            @pl.when(ck + NBUF - 1 < CPW)
            def _():
                @pl.when(ck >= 1)
                def _():
                    odesc(ck - 1, pn).wait()
                gstart(ck + NBUF - 1, pn)

            gwait(ck, p)
            off = lax.rem(ck * CHUNK, MAXLEN)
            buf = bufs[p]

            @pl.loop(0, CHUNK, step=2)
            def _(r):
                for rr in range(2):
                    for g in range(EMBED // 16):
                        s = pl.ds(g * 16, 16)
                        plsc.addupdate(buf.at[r + rr, s], pos_v[off + r + rr, s])

            odesc(ck, p).start()

    for k in range(NBUF):
        odesc(CPW - NBUF + k, k).wait()


@jax.jit
def kernel(x, token_table, pos_table):
    x2 = x.reshape(NCHUNKS, CHUNK).astype(jnp.int32)
    pos2 = jnp.concatenate([pos_table, pos_table], axis=0)

    mesh = plsc.VectorSubcoreMesh(core_axis_name="c", subcore_axis_name="s")
    run = pl.kernel(
        _emb_body,
        out_type=jax.ShapeDtypeStruct((BFLAT, EMBED), jnp.float32),
        mesh=mesh,
        scratch_types=(
            [pltpu.VMEM((CPW, CHUNK), jnp.int32),
             pltpu.VMEM((2 * MAXLEN, EMBED), jnp.float32)]
            + [pltpu.VMEM((CHUNK, EMBED), jnp.float32)] * NBUF
            + [pltpu.SemaphoreType.DMA] * (2 * NBUF)
        ),
        compiler_params=pltpu.CompilerParams(use_tc_tiling_on_sc=False),
    )
    out = run(x2, token_table, pos2)
    return out.reshape(BATCH, MAXLEN, EMBED)


# speedup vs baseline: 3.9721x; 1.3998x over previous
import functools

import jax
import jax.numpy as jnp
from jax import lax
from jax.experimental import pallas as pl
from jax.experimental.pallas import tpu as pltpu
from jax.experimental.pallas import tpu_sc as plsc

NC = 2
NS = 16
NW = NC * NS
CHUNK = 128
MAXLEN = 200
EMBED = 64
BATCH = 4096
BFLAT = BATCH * MAXLEN
NCHUNKS = BFLAT // CHUNK
CPW = NCHUNKS // NW


NBUF = 4


def _emb_body(x2_hbm, tab_hbm, pos2_hbm, out_hbm, idx_v, pos_v,
              b0, b1, b2, b3, g0, g1, g2, g3, o0, o1, o2, o3):
    bufs = (b0, b1, b2, b3)
    gsems = (g0, g1, g2, g3)
    osems = (o0, o1, o2, o3)

    wid = lax.axis_index("s") * NC + lax.axis_index("c")
    row0 = wid * CPW
    flat0 = row0 * CHUNK

    pltpu.sync_copy(pos2_hbm, pos_v)
    pltpu.sync_copy(x2_hbm.at[pl.ds(row0, CPW)], idx_v)

    def gstart(c, p):
        pltpu.make_async_copy(tab_hbm.at[idx_v.at[c]], bufs[p], gsems[p]).start()

    def gwait(c, p):
        pltpu.make_async_copy(tab_hbm.at[idx_v.at[c]], bufs[p], gsems[p]).wait()

    def odesc(c, p):
        dst = out_hbm.at[pl.ds(flat0 + c * CHUNK, CHUNK)]
        return pltpu.make_async_copy(bufs[p], dst, osems[p])

    for p in range(NBUF - 1):
        gstart(p, p)

    @pl.loop(0, CPW, step=NBUF)
    def _(c):
        for k in range(NBUF):
            ck = c + k
            p = k
            pn = (k + NBUF - 1) % NBUF

            @pl.when(ck + NBUF - 1 < CPW)
            def _():
                @pl.when(ck >= 1)
                def _():
                    odesc(ck - 1, pn).wait()
                gstart(ck + NBUF - 1, pn)

            gwait(ck, p)
            off = lax.rem(ck * CHUNK, MAXLEN)
            buf = bufs[p]

            @plsc.parallel_loop(0, CHUNK, unroll=4)
            def _(r):
                for g in range(EMBED // 16):
                    s = pl.ds(g * 16, 16)
                    plsc.addupdate(buf.at[r, s], pos_v[off + r, s])

            odesc(ck, p).start()

    for k in range(NBUF):
        odesc(CPW - NBUF + k, k).wait()


@jax.jit
def kernel(x, token_table, pos_table):
    x2 = x.reshape(NCHUNKS, CHUNK).astype(jnp.int32)
    pos2 = jnp.concatenate([pos_table, pos_table], axis=0)

    mesh = plsc.VectorSubcoreMesh(core_axis_name="c", subcore_axis_name="s")
    run = pl.kernel(
        _emb_body,
        out_type=jax.ShapeDtypeStruct((BFLAT, EMBED), jnp.float32),
        mesh=mesh,
        scratch_types=(
            [pltpu.VMEM((CPW, CHUNK), jnp.int32),
             pltpu.VMEM((2 * MAXLEN, EMBED), jnp.float32)]
            + [pltpu.VMEM((CHUNK, EMBED), jnp.float32)] * NBUF
            + [pltpu.SemaphoreType.DMA] * (2 * NBUF)
        ),
        compiler_params=pltpu.CompilerParams(use_tc_tiling_on_sc=False),
    )
    out = run(x2, token_table, pos2)
    return out.reshape(BATCH, MAXLEN, EMBED)
